# baseline (speedup 1.0000x reference)
import jax
import jax.numpy as jnp
from jax.experimental import pallas as pl
from jax.experimental.pallas import tpu as pltpu

E = 160000
BLOCK = 6400
NB = E // BLOCK


def _main_kernel(cat_ref, xn_ref, a_ref, w_ref, bout_ref,
                 out_ref, m_ref):
    @pl.when(pl.program_id(0) == 0)
    def _fold():
        acc = jnp.zeros((128, 512), dtype=jnp.float32)
        for k in range(4):
            acc = acc + jax.lax.dot_general(
                a_ref[k], w_ref[k],
                dimension_numbers=(((1,), (0,)), ((), ())),
                preferred_element_type=jnp.float32,
            )
        row = jax.lax.broadcasted_iota(jnp.int32, (128, 512), 0)
        m_ref[...] = acc + jnp.where(row == 90, bout_ref[...], 0.0)

    b = out_ref.shape[0]
    l = jax.lax.broadcasted_iota(jnp.int32, (112, b), 0)
    i0 = cat_ref[0, 0]
    i1 = cat_ref[1, 0]
    i2 = cat_ref[2, 0]
    oh = ((l == i0) | (l == i1 + 50) | (l == i2 + 70) | (l == 74)).astype(
        jnp.float32
    )
    lhs_t = jnp.concatenate([xn_ref[...], oh], axis=0)
    out_ref[...] = jax.lax.dot_general(
        lhs_t, m_ref[...],
        dimension_numbers=(((0,), (0,)), ((), ())),
        preferred_element_type=jnp.float32,
    )


def kernel(edge_attr_cat, edge_attr_num, emb_acc, emb_trans, emb_season,
           W_num, b_num, W_out, b_out):
    f32 = jnp.float32
    z = lambda n: jnp.zeros((n, 128), dtype=f32)
    a0 = jnp.concatenate([z(16), emb_acc.astype(f32), z(62)], axis=0)
    a1 = jnp.concatenate([z(66), emb_trans.astype(f32), z(42)], axis=0)
    a2 = jnp.concatenate([z(86), emb_season.astype(f32), z(38)], axis=0)
    a3 = jnp.concatenate(
        [W_num.astype(f32), z(74), b_num.astype(f32)[None, :], z(37)], axis=0
    )
    astack = jnp.stack([a0, a1, a2, a3], axis=0)
    w_blocks = W_out.astype(f32).reshape(4, 128, 512)

    cat_t = edge_attr_cat.astype(jnp.int32).T.reshape(3, NB, 1, BLOCK)
    xnum_t = edge_attr_num.astype(f32).T

    out = pl.pallas_call(
        _main_kernel,
        grid=(NB,),
        in_specs=[
            pl.BlockSpec((3, 1, 1, BLOCK), lambda i: (0, i, 0, 0)),
            pl.BlockSpec((16, BLOCK), lambda i: (0, i)),
            pl.BlockSpec((4, 128, 128), lambda i: (0, 0, 0)),
            pl.BlockSpec((4, 128, 512), lambda i: (0, 0, 0)),
            pl.BlockSpec((1, 512), lambda i: (0, 0)),
        ],
        out_specs=pl.BlockSpec((BLOCK, 512), lambda i: (i, 0)),
        out_shape=jax.ShapeDtypeStruct((E, 512), f32),
        scratch_shapes=[pltpu.VMEM((128, 512), f32)],
        compiler_params=pltpu.CompilerParams(
            dimension_semantics=("arbitrary",),
        ),
    )(cat_t, xnum_t, astack, w_blocks, b_out.astype(f32)[None, :])
    return out
